# baseline (speedup 1.0000x reference)
import functools

import jax
import jax.numpy as jnp
from jax import lax
from jax.experimental import pallas as pl
from jax.experimental.pallas import tpu as pltpu
from jax.experimental.pallas import tpu_sc as plsc

NC = 2
NS = 16
NW = NC * NS

KCH = 72
DEPTH = 5
LEAD = 3


def _tc1_body(x_ref, w_ref, b_ref, xt_ref, xh_ref):
    xt = lax.dot_general(x_ref[...], w_ref[...], (((1,), (1,)), ((), ())),
                         preferred_element_type=jnp.float32,
                         precision=lax.Precision.HIGHEST)
    xt = xt + b_ref[...]
    xt_ref[...] = xt
    xh_ref[...] = xt * 0.5


def _tc1(x, W_lin, b_lin2d):
    N, D = x.shape
    H = W_lin.shape[0]
    BN = 1000
    grid = (N // BN,)
    return pl.pallas_call(
        _tc1_body,
        grid=grid,
        in_specs=[
            pl.BlockSpec((BN, D), lambda i: (i, 0)),
            pl.BlockSpec((H, D), lambda i: (0, 0)),
            pl.BlockSpec((1, H), lambda i: (0, 0)),
        ],
        out_specs=[
            pl.BlockSpec((BN, H), lambda i: (i, 0)),
            pl.BlockSpec((BN, H), lambda i: (i, 0)),
        ],
        out_shape=[
            jax.ShapeDtypeStruct((N, H), jnp.float32),
            jax.ShapeDtypeStruct((N, H), jnp.float32),
        ],
    )(x, W_lin, b_lin2d)


def _tc2_body(c0_ref, c1_ref, att_ref, w1_ref, b1_ref, a_ref, w2_ref,
              b2_ref, ho_ref, hp_ref):
    c = c0_ref[...] + c1_ref[...]
    logits = jnp.sum(c * att_ref[...], axis=1, keepdims=True)
    gate = jax.nn.sigmoid(logits)
    ho = gate * c
    ho_ref[...] = ho
    h1 = lax.dot_general(ho, w1_ref[...], (((1,), (1,)), ((), ())),
                         preferred_element_type=jnp.float32,
                         precision=lax.Precision.HIGHEST) + b1_ref[...]
    h1 = jnp.where(h1 >= 0, h1, h1 * a_ref[...])
    hp_ref[...] = lax.dot_general(h1, w2_ref[...], (((1,), (1,)), ((), ())),
                                  preferred_element_type=jnp.float32,
                                  precision=lax.Precision.HIGHEST) + b2_ref[...]


def _tc2(c_all, N, N_pad, attention, W1, b1_2d, a2d, W2, b2_2d):
    H = c_all.shape[1]
    BN = N_pad // NS
    assert N_pad % BN == 0
    grid = ((N + BN - 1) // BN,)
    off_blocks = N_pad // BN
    full = lambda i: (0, 0)
    blk = lambda i: (i, 0)
    blk1 = lambda i: (i + off_blocks, 0)
    return pl.pallas_call(
        _tc2_body,
        grid=grid,
        in_specs=[
            pl.BlockSpec((BN, H), blk),
            pl.BlockSpec((BN, H), blk1),
            pl.BlockSpec((1, H), full),
            pl.BlockSpec((H, H), full),
            pl.BlockSpec((1, H), full),
            pl.BlockSpec((1, 1), full),
            pl.BlockSpec((H, H), full),
            pl.BlockSpec((1, H), full),
        ],
        out_specs=[
            pl.BlockSpec((BN, H), blk),
            pl.BlockSpec((BN, H), blk),
        ],
        out_shape=[
            jax.ShapeDtypeStruct((N, H), jnp.float32),
            jax.ShapeDtypeStruct((N, H), jnp.float32),
        ],
    )(c_all, c_all, attention, W1, b1_2d, a2d, W2, b2_2d)


def _phase(src_hbm, xc_hbm, acc, idx_v, rows_v, gsem, ssem, w, T, K):
    R = T // DEPTH
    assert T % DEPTH == 0 and R >= 2

    def load_round(r, r3):
        base = pl.multiple_of(w * T + r * DEPTH, DEPTH)
        pltpu.sync_copy(xc_hbm.at[pl.ds(base, DEPTH)], idx_v.at[r3])

    def slot(q):
        return rows_v.at[pl.ds(q * K, K)]

    def fire_gather(r3, i, q):
        pltpu.async_copy(src_hbm.at[idx_v.at[r3, i, 0]], slot(q), gsem.at[q])

    def wait_gather(r3, i, q):
        pltpu.make_async_copy(src_hbm.at[idx_v.at[r3, i, 0]], slot(q),
                              gsem.at[q]).wait()

    def fire_scatter(r3, i, q):
        pltpu.async_copy(slot(q), acc.at[idx_v.at[r3, i, 1]], ssem.at[q],
                         add=True)

    def wait_scatter(r3, i, q):
        pltpu.make_async_copy(slot(q), acc.at[idx_v.at[r3, i, 1]],
                              ssem.at[q]).wait()

    def visit(rm3, q, first, last):
        if not last:
            qa = (q + LEAD) % DEPTH
            if q < DEPTH - LEAD:
                rg3 = rm3
            else:
                rg3 = _rem3(rm3 + 1)
            if not (first and q < DEPTH - LEAD):
                wait_scatter(rm3, q, qa)
            fire_gather(rg3, qa, qa)
        wait_gather(rm3, q, q)
        fire_scatter(rm3, q, q)

    def _rem3(v):
        if isinstance(v, int):
            return v % 3
        return lax.rem(v, 3)

    load_round(0, 0)
    load_round(1, 1)
    for q in range(LEAD):
        fire_gather(0, q, q)
    for q in range(DEPTH):
        visit(0, q, True, False)

    def round_body(r, carry):
        rm3 = lax.rem(r, 3)
        load_round(r + 1, lax.rem(r + 1, 3))
        for q in range(DEPTH):
            visit(rm3, q, False, False)
        return carry

    lax.fori_loop(1, R - 1, round_body, 0)

    for q in range(DEPTH):
        visit((R - 1) % 3, q, False, q >= DEPTH - LEAD)
    for q in range(DEPTH):
        wait_scatter(0, 0, q)


def _make_sc_upper(N_pad, H, T):
    RPT = N_pad // NS
    K = KCH
    mesh = plsc.VectorSubcoreMesh(core_axis_name="c", subcore_axis_name="s",
                                  num_cores=NC, num_subcores=NS)

    @functools.partial(
        pl.kernel,
        out_type=jax.ShapeDtypeStruct((2 * N_pad, H), jnp.float32),
        mesh=mesh,
        scratch_types=[
            pltpu.VMEM_SHARED((N_pad, H), jnp.float32),
            pltpu.VMEM((3, DEPTH, 2, K), jnp.int32),
            pltpu.VMEM((DEPTH * K, H), jnp.float32),
            pltpu.SemaphoreType.DMA((DEPTH,)),
            pltpu.SemaphoreType.DMA((DEPTH,)),
        ],
    )
    def sc_upper(xh_hbm, xc_hbm, z_hbm, out, acc, idx_v, rows_v, gsem, ssem):
        c = lax.axis_index("c")
        s = lax.axis_index("s")
        w = s * NC + c
        pltpu.sync_copy(z_hbm, acc.at[pl.ds(s * RPT, RPT)])
        plsc.subcore_barrier()
        _phase(xh_hbm, xc_hbm, acc, idx_v, rows_v, gsem, ssem, w, T, K)
        plsc.subcore_barrier()
        off = pl.multiple_of(c * N_pad + s * RPT, 8)
        pltpu.sync_copy(acc.at[pl.ds(s * RPT, RPT)], out.at[pl.ds(off, RPT)])

    return sc_upper


def _make_sc_combine(N_pad, H, TE, TB):
    RPT = N_pad // NS
    K = KCH
    mesh = plsc.VectorSubcoreMesh(core_axis_name="c", subcore_axis_name="s",
                                  num_cores=NC, num_subcores=NS)

    @functools.partial(
        pl.kernel,
        out_type=jax.ShapeDtypeStruct((2 * N_pad, H), jnp.float32),
        mesh=mesh,
        scratch_types=[
            pltpu.VMEM_SHARED((N_pad, H), jnp.float32),
            pltpu.VMEM((3, DEPTH, 2, K), jnp.int32),
            pltpu.VMEM((DEPTH * K, H), jnp.float32),
            pltpu.SemaphoreType.DMA((DEPTH,)),
            pltpu.SemaphoreType.DMA((DEPTH,)),
        ],
    )
    def sc_combine(xt_hbm, xce_hbm, up_hbm, xcb_hbm, z_hbm, out,
                   acc, idx_v, rows_v, gsem, ssem):
        c = lax.axis_index("c")
        s = lax.axis_index("s")
        w = s * NC + c
        pltpu.sync_copy(z_hbm, acc.at[pl.ds(s * RPT, RPT)])
        plsc.subcore_barrier()
        _phase(xt_hbm, xce_hbm, acc, idx_v, rows_v, gsem, ssem, w, TE, K)
        _phase(up_hbm, xcb_hbm, acc, idx_v, rows_v, gsem, ssem, w, TB, K)
        plsc.subcore_barrier()
        off = pl.multiple_of(c * N_pad + s * RPT, 8)
        pltpu.sync_copy(acc.at[pl.ds(s * RPT, RPT)], out.at[pl.ds(off, RPT)])

    return sc_combine


def _pad_and_order(gidx, sidx, trash):
    K = KCH
    ne = gidx.shape[0]
    quantum = NW * K * DEPTH
    ne_pad = max(2, -(-ne // quantum)) * quantum
    pad = ne_pad - ne
    gp = jnp.concatenate([gidx, jnp.zeros((pad,), jnp.int32)])
    sp = jnp.concatenate([sidx, jnp.full((pad,), trash, jnp.int32)])
    C = ne_pad // K
    T = C // NW
    xc = jnp.stack([gp.reshape(C, K), sp.reshape(C, K)], axis=1)
    order = (jnp.arange(NW)[:, None] + jnp.arange(T)[None, :] * NW).reshape(-1)
    return xc[order], T


def kernel(x, adj_indices, bup_indices, W_lin, b_lin, attention, W1, b1,
           prelu_a, W2, b2):
    N, D = x.shape
    H = W_lin.shape[0]

    RPT = ((N + NS - 1) // NS + 7) // 8 * 8
    N_pad = NS * RPT
    zeros = jnp.zeros((RPT, H), jnp.float32)
    trash = N_pad - 1

    xce, TE = _pad_and_order(adj_indices[1], adj_indices[0], trash)
    xcb_up, TU = _pad_and_order(bup_indices[1], bup_indices[0], trash)
    xcb0, TB0 = _pad_and_order(bup_indices[0], bup_indices[1], trash)
    xcb1 = xcb0.at[:, 0, :].add(jnp.int32(N_pad))
    K = KCH
    xcb = jnp.concatenate([xcb0.reshape(NW, TB0, 2, K),
                           xcb1.reshape(NW, TB0, 2, K)], axis=1)
    xcb = xcb.reshape(NW * 2 * TB0, 2, K)
    TB = 2 * TB0

    x_trans, x_half = _tc1(x, W_lin, b_lin.reshape(1, H))

    up_p = _make_sc_upper(N_pad, H, TU)(x_half, xcb_up, zeros)

    c_all = _make_sc_combine(N_pad, H, TE, TB)(
        x_trans, xce, up_p, xcb, zeros)

    h_online, h_pred = _tc2(c_all, N, N_pad, attention, W1, b1.reshape(1, H),
                            prelu_a.reshape(1, 1), W2, b2.reshape(1, H))
    return (h_online, h_pred, h_online)


# speedup vs baseline: 2.1238x; 2.1238x over previous
import functools

import jax
import jax.numpy as jnp
from jax import lax
from jax.experimental import pallas as pl
from jax.experimental.pallas import tpu as pltpu
from jax.experimental.pallas import tpu_sc as plsc

NC = 2
NS = 16
NW = NC * NS

KCH = 64
DEPTH = 5
LEAD = 3


def _tc1_body(x_ref, w_ref, b_ref, xt_ref, xh_ref):
    xt = lax.dot_general(x_ref[...], w_ref[...], (((1,), (1,)), ((), ())),
                         preferred_element_type=jnp.float32,
                         precision=lax.Precision.HIGHEST)
    xt = xt + b_ref[...]
    xt_ref[...] = xt
    xh_ref[...] = xt * 0.5


def _tc1(x, W_lin, b_lin2d):
    N, D = x.shape
    H = W_lin.shape[0]
    BN = 1000
    grid = (N // BN,)
    return pl.pallas_call(
        _tc1_body,
        grid=grid,
        in_specs=[
            pl.BlockSpec((BN, D), lambda i: (i, 0)),
            pl.BlockSpec((H, D), lambda i: (0, 0)),
            pl.BlockSpec((1, H), lambda i: (0, 0)),
        ],
        out_specs=[
            pl.BlockSpec((BN, H), lambda i: (i, 0)),
            pl.BlockSpec((BN, H), lambda i: (i, 0)),
        ],
        out_shape=[
            jax.ShapeDtypeStruct((N, H), jnp.float32),
            jax.ShapeDtypeStruct((N, H), jnp.float32),
        ],
    )(x, W_lin, b_lin2d)


def _tc2_body(c0_ref, c1_ref, att_ref, w1_ref, b1_ref, a_ref, w2_ref,
              b2_ref, ho_ref, hp_ref):
    c = c0_ref[...] + c1_ref[...]
    logits = jnp.sum(c * att_ref[...], axis=1, keepdims=True)
    gate = jax.nn.sigmoid(logits)
    ho = gate * c
    ho_ref[...] = ho
    h1 = lax.dot_general(ho, w1_ref[...], (((1,), (1,)), ((), ())),
                         preferred_element_type=jnp.float32,
                         precision=lax.Precision.HIGHEST) + b1_ref[...]
    h1 = jnp.where(h1 >= 0, h1, h1 * a_ref[...])
    hp_ref[...] = lax.dot_general(h1, w2_ref[...], (((1,), (1,)), ((), ())),
                                  preferred_element_type=jnp.float32,
                                  precision=lax.Precision.HIGHEST) + b2_ref[...]


def _tc2(c_all, N, N_pad, attention, W1, b1_2d, a2d, W2, b2_2d):
    H = c_all.shape[1]
    BN = N_pad // NS
    assert N_pad % BN == 0
    grid = ((N + BN - 1) // BN,)
    off_blocks = N_pad // BN
    full = lambda i: (0, 0)
    blk = lambda i: (i, 0)
    blk1 = lambda i: (i + off_blocks, 0)
    return pl.pallas_call(
        _tc2_body,
        grid=grid,
        in_specs=[
            pl.BlockSpec((BN, H), blk),
            pl.BlockSpec((BN, H), blk1),
            pl.BlockSpec((1, H), full),
            pl.BlockSpec((H, H), full),
            pl.BlockSpec((1, H), full),
            pl.BlockSpec((1, 1), full),
            pl.BlockSpec((H, H), full),
            pl.BlockSpec((1, H), full),
        ],
        out_specs=[
            pl.BlockSpec((BN, H), blk),
            pl.BlockSpec((BN, H), blk),
        ],
        out_shape=[
            jax.ShapeDtypeStruct((N, H), jnp.float32),
            jax.ShapeDtypeStruct((N, H), jnp.float32),
        ],
    )(c_all, c_all, attention, W1, b1_2d, a2d, W2, b2_2d)


def _sum_body(a_ref, b_ref, o_ref):
    o_ref[...] = a_ref[...] + b_ref[...]


def _tc_sum(up_p, N_pad):
    H = up_p.shape[1]
    BN = N_pad // NS
    grid = (NS,)
    return pl.pallas_call(
        _sum_body,
        grid=grid,
        in_specs=[
            pl.BlockSpec((BN, H), lambda i: (i, 0)),
            pl.BlockSpec((BN, H), lambda i: (i + NS, 0)),
        ],
        out_specs=pl.BlockSpec((BN, H), lambda i: (i, 0)),
        out_shape=jax.ShapeDtypeStruct((N_pad, H), jnp.float32),
    )(up_p, up_p)


def _phase(src_hbm, xc_hbm, acc, idx_v, rows_v, gsem, ssem, w, T, K):
    R = T // DEPTH
    assert T % DEPTH == 0 and R >= 2

    def load_round(r, r3):
        base = pl.multiple_of(w * T + r * DEPTH, DEPTH)
        pltpu.sync_copy(xc_hbm.at[pl.ds(base, DEPTH)], idx_v.at[r3])

    def slot(q):
        return rows_v.at[pl.ds(q * K, K)]

    def fire_gather(r3, i, q):
        pltpu.async_copy(src_hbm.at[idx_v.at[r3, i, 0]], slot(q), gsem.at[q])

    def wait_gather(r3, i, q):
        pltpu.make_async_copy(src_hbm.at[idx_v.at[r3, i, 0]], slot(q),
                              gsem.at[q]).wait()

    def fire_scatter(r3, i, q):
        pltpu.async_copy(slot(q), acc.at[idx_v.at[r3, i, 1]], ssem.at[q],
                         add=True)

    def wait_scatter(r3, i, q):
        pltpu.make_async_copy(slot(q), acc.at[idx_v.at[r3, i, 1]],
                              ssem.at[q]).wait()

    def visit(rm3, q, first, last):
        if not last:
            qa = (q + LEAD) % DEPTH
            if q < DEPTH - LEAD:
                rg3 = rm3
            else:
                rg3 = _rem3(rm3 + 1)
            if not (first and q < DEPTH - LEAD):
                wait_scatter(rm3, q, qa)
            fire_gather(rg3, qa, qa)
        wait_gather(rm3, q, q)
        fire_scatter(rm3, q, q)

    def _rem3(v):
        if isinstance(v, int):
            return v % 3
        return lax.rem(v, 3)

    load_round(0, 0)
    load_round(1, 1)
    for q in range(LEAD):
        fire_gather(0, q, q)
    for q in range(DEPTH):
        visit(0, q, True, False)

    def round_body(r, carry):
        rm3 = lax.rem(r, 3)
        load_round(r + 1, lax.rem(r + 1, 3))
        for q in range(DEPTH):
            visit(rm3, q, False, False)
        return carry

    lax.fori_loop(1, R - 1, round_body, 0)

    for q in range(DEPTH):
        visit((R - 1) % 3, q, False, q >= DEPTH - LEAD)
    for q in range(DEPTH):
        wait_scatter(0, 0, q)


def _make_sc_upper(N_pad, H, T):
    RPT = N_pad // NS
    K = KCH
    mesh = plsc.VectorSubcoreMesh(core_axis_name="c", subcore_axis_name="s",
                                  num_cores=NC, num_subcores=NS)

    @functools.partial(
        pl.kernel,
        out_type=jax.ShapeDtypeStruct((2 * N_pad, H), jnp.float32),
        mesh=mesh,
        scratch_types=[
            pltpu.VMEM_SHARED((N_pad, H), jnp.float32),
            pltpu.VMEM((3, DEPTH, 2, K), jnp.int32),
            pltpu.VMEM((DEPTH * K, H), jnp.float32),
            pltpu.SemaphoreType.DMA((DEPTH,)),
            pltpu.SemaphoreType.DMA((DEPTH,)),
        ],
    )
    def sc_upper(xh_hbm, xc_hbm, z_hbm, out, acc, idx_v, rows_v, gsem, ssem):
        c = lax.axis_index("c")
        s = lax.axis_index("s")
        w = s * NC + c
        pltpu.sync_copy(z_hbm, acc.at[pl.ds(s * RPT, RPT)])
        plsc.subcore_barrier()
        _phase(xh_hbm, xc_hbm, acc, idx_v, rows_v, gsem, ssem, w, T, K)
        plsc.subcore_barrier()
        off = pl.multiple_of(c * N_pad + s * RPT, 8)
        pltpu.sync_copy(acc.at[pl.ds(s * RPT, RPT)], out.at[pl.ds(off, RPT)])

    return sc_upper


def _make_sc_combine(N_pad, H, TE, TB):
    RPT = N_pad // NS
    K = KCH
    mesh = plsc.VectorSubcoreMesh(core_axis_name="c", subcore_axis_name="s",
                                  num_cores=NC, num_subcores=NS)

    @functools.partial(
        pl.kernel,
        out_type=jax.ShapeDtypeStruct((2 * N_pad, H), jnp.float32),
        mesh=mesh,
        scratch_types=[
            pltpu.VMEM_SHARED((N_pad, H), jnp.float32),
            pltpu.VMEM((3, DEPTH, 2, K), jnp.int32),
            pltpu.VMEM((DEPTH * K, H), jnp.float32),
            pltpu.SemaphoreType.DMA((DEPTH,)),
            pltpu.SemaphoreType.DMA((DEPTH,)),
        ],
    )
    def sc_combine(xt_hbm, xce_hbm, up_hbm, xcb_hbm, z_hbm, out,
                   acc, idx_v, rows_v, gsem, ssem):
        c = lax.axis_index("c")
        s = lax.axis_index("s")
        w = s * NC + c
        pltpu.sync_copy(z_hbm, acc.at[pl.ds(s * RPT, RPT)])
        plsc.subcore_barrier()
        _phase(xt_hbm, xce_hbm, acc, idx_v, rows_v, gsem, ssem, w, TE, K)
        _phase(up_hbm, xcb_hbm, acc, idx_v, rows_v, gsem, ssem, w, TB, K)
        plsc.subcore_barrier()
        off = pl.multiple_of(c * N_pad + s * RPT, 8)
        pltpu.sync_copy(acc.at[pl.ds(s * RPT, RPT)], out.at[pl.ds(off, RPT)])

    return sc_combine


def _pad_and_order(gidx, sidx, trash):
    K = KCH
    ne = gidx.shape[0]
    quantum = NW * K * DEPTH
    ne_pad = max(2, -(-ne // quantum)) * quantum
    pad = ne_pad - ne
    gp = jnp.concatenate([gidx, jnp.zeros((pad,), jnp.int32)])
    sp = jnp.concatenate([sidx, jnp.full((pad,), trash, jnp.int32)])
    C = ne_pad // K
    T = C // NW
    xc = jnp.stack([gp.reshape(C, K), sp.reshape(C, K)], axis=1)
    order = (jnp.arange(NW)[:, None] + jnp.arange(T)[None, :] * NW).reshape(-1)
    return xc[order], T


def kernel(x, adj_indices, bup_indices, W_lin, b_lin, attention, W1, b1,
           prelu_a, W2, b2):
    N, D = x.shape
    H = W_lin.shape[0]

    RPT = ((N + NS - 1) // NS + 7) // 8 * 8
    N_pad = NS * RPT
    zeros = jnp.zeros((RPT, H), jnp.float32)
    trash = N_pad - 1

    xce, TE = _pad_and_order(adj_indices[1], adj_indices[0], trash)
    xcb_up, TU = _pad_and_order(bup_indices[1], bup_indices[0], trash)
    xcb, TB = _pad_and_order(bup_indices[0], bup_indices[1], trash)

    x_trans, x_half = _tc1(x, W_lin, b_lin.reshape(1, H))

    up_p = _make_sc_upper(N_pad, H, TU)(x_half, xcb_up, zeros)
    up_sum = _tc_sum(up_p, N_pad)

    c_all = _make_sc_combine(N_pad, H, TE, TB)(
        x_trans, xce, up_sum, xcb, zeros)

    h_online, h_pred = _tc2(c_all, N, N_pad, attention, W1, b1.reshape(1, H),
                            prelu_a.reshape(1, 1), W2, b2.reshape(1, H))
    return (h_online, h_pred, h_online)
